# Initial kernel scaffold; baseline (speedup 1.0000x reference)
import jax
import jax.numpy as jnp
from jax.experimental import pallas as pl


def kernel(inputs, labels, hyp, no_hyp, neighbors, color_ids):
    raise NotImplementedError("write your pallas kernel here")


# speedup vs baseline: 3.0768x; 3.0768x over previous
import jax
import jax.numpy as jnp
from jax.experimental import pallas as pl
from jax.experimental.pallas import tpu as pltpu

P = 50
B = 128
N = 784
DEG = 4
LL = 2
MAX_ITER_SEG = 3
NUM_COLORS = 2
KTOT = MAX_ITER_SEG + LL
DP = float(DEG)

BC = 16


def _dadmm_kernel(nbr_ref, cid_ref, hs_ref, x_ref, a0_ref, om0_ref, lab_ref,
                  a_out, om_out, a_s, om_s, mu_s, suma_s, sumom_s):
    a_s[...] = a0_ref[...]
    om_s[...] = om0_ref[...]
    mu_s[...] = jnp.zeros((P, BC, N), jnp.float32)
    lam = jnp.zeros((P, BC, 1), jnp.float32)
    x = x_ref[...]
    lab = lab_ref[...]
    cid = cid_ref[...]

    def nsum(p, carry):
        n0 = nbr_ref[p, 0]
        n1 = nbr_ref[p, 1]
        n2 = nbr_ref[p, 2]
        n3 = nbr_ref[p, 3]
        suma_s[p] = a_s[n0] + a_s[n1] + a_s[n2] + a_s[n3]
        sumom_s[p] = om_s[n0] + om_s[n1] + om_s[n2] + om_s[n3]
        return carry

    def neighbor_sums():
        jax.lax.fori_loop(0, P, nsum, 0)

    neighbor_sums()
    for k in range(KTOT):
        h = jnp.abs(hs_ref[k])
        h0 = h[:, 0:1][:, :, None]
        h1 = h[:, 1:2][:, :, None]
        h2 = h[:, 2:3][:, :, None]
        h3 = h[:, 3:4][:, :, None]
        h4 = h[:, 4:5][:, :, None]
        h5 = h[:, 5:6][:, :, None]
        for color in range(NUM_COLORS):
            a = a_s[...]
            om = om_s[...]
            suma = suma_s[...]
            sumom = sumom_s[...]
            s = jnp.sum(x * a, axis=2, keepdims=True)
            c = s + om - lab
            grad_a = x * c + (h0 * DP) * a + DP * mu_s[...] - h0 * suma
            a_new = a - h1 * grad_a
            grad_om = c + (h2 * DP) * om + DP * lam - h2 * sumom
            om_new = om - h5 * grad_om
            mask = (cid == color)[:, :, None]
            a_s[...] = jnp.where(mask, a_new, a)
            om_s[...] = jnp.where(mask, om_new, om)
            neighbor_sums()
        mu_s[...] = mu_s[...] + h3 * (DP * a_s[...] - suma_s[...])
        lam = lam + h4 * (DP * om_s[...] - sumom_s[...])
    a_out[...] = a_s[...]
    om_out[...] = om_s[...]


def kernel(inputs, labels, hyp, no_hyp, neighbors, color_ids):
    x = inputs.reshape(P, B, N)
    lab = labels.reshape(P, B, 1)
    hs = jnp.concatenate([no_hyp, hyp], axis=0)
    nbr = neighbors.astype(jnp.int32)
    cid = color_ids.astype(jnp.int32).reshape(P, 1)

    kinit = jax.random.key(1234)
    ka, ko = jax.random.split(kinit)
    a0 = jax.random.normal(ka, (P, B, N, 1), dtype=jnp.float32).reshape(P, B, N)
    om0 = jax.random.uniform(ko, (P, B, 1, 1), dtype=jnp.float32).reshape(P, B, 1)

    grid = B // BC
    big = lambda i: (0, i, 0)
    a_out, om_out = pl.pallas_call(
        _dadmm_kernel,
        grid=(grid,),
        in_specs=[
            pl.BlockSpec(memory_space=pltpu.SMEM),
            pl.BlockSpec((P, 1), lambda i: (0, 0)),
            pl.BlockSpec((KTOT, P, 6), lambda i: (0, 0, 0)),
            pl.BlockSpec((P, BC, N), big),
            pl.BlockSpec((P, BC, N), big),
            pl.BlockSpec((P, BC, 1), big),
            pl.BlockSpec((P, BC, 1), big),
        ],
        out_specs=[
            pl.BlockSpec((P, BC, N), big),
            pl.BlockSpec((P, BC, 1), big),
        ],
        out_shape=[
            jax.ShapeDtypeStruct((P, B, N), jnp.float32),
            jax.ShapeDtypeStruct((P, B, 1), jnp.float32),
        ],
        scratch_shapes=[
            pltpu.VMEM((P, BC, N), jnp.float32),
            pltpu.VMEM((P, BC, 1), jnp.float32),
            pltpu.VMEM((P, BC, N), jnp.float32),
            pltpu.VMEM((P, BC, N), jnp.float32),
            pltpu.VMEM((P, BC, 1), jnp.float32),
        ],
        compiler_params=pltpu.CompilerParams(
            dimension_semantics=("arbitrary",),
        ),
    )(nbr, cid, hs, x, a0, om0, lab)
    return a_out.reshape(P, B, N, 1), om_out.reshape(P, B, 1, 1)
